# baseline (speedup 1.0000x reference)
import functools

import jax
import jax.numpy as jnp
from jax import lax
from jax.experimental import pallas as pl
from jax.experimental.pallas import tpu as pltpu
from jax.experimental.pallas import tpu_sc as plsc

_NC = 2
_NS = 16
_NW = _NC * _NS

_CHUNK = 8
_NBUF = 4


@functools.partial(jax.jit, static_argnames=("n_tokens", "hidden"))
def _embed_lookup(ids_flat, table, *, n_tokens, hidden):
    per_w = n_tokens // _NW
    n_chunks = per_w // _CHUNK

    mesh = plsc.VectorSubcoreMesh(core_axis_name="c", subcore_axis_name="s")

    @functools.partial(
        pl.kernel,
        out_type=jax.ShapeDtypeStruct((n_tokens, hidden), jnp.float32),
        mesh=mesh,
        scratch_types=(
            [pltpu.VMEM((per_w,), jnp.int32)]
            + [pltpu.VMEM((_CHUNK, hidden), jnp.float32)] * _NBUF
            + [pltpu.SemaphoreType.DMA] * (2 * _NBUF)
        ),
    )
    def k(table_hbm, idx_hbm, out_hbm, idx_v, *rest):
        bufs = rest[:_NBUF]
        gsems = rest[_NBUF:2 * _NBUF]
        wsems = rest[2 * _NBUF:]

        wid = lax.axis_index("s") * _NC + lax.axis_index("c")
        base = wid * per_w

        pltpu.sync_copy(idx_hbm.at[pl.ds(base, per_w)], idx_v)

        def gather(c, b):
            pltpu.async_copy(
                table_hbm.at[idx_v.at[pl.ds(c * _CHUNK, _CHUNK)]],
                bufs[b], gsems[b])

        def wait_gather(b):
            pltpu.make_async_copy(
                table_hbm.at[idx_v.at[pl.ds(0, _CHUNK)]],
                bufs[b], gsems[b]).wait()

        def writeback(c, b):
            pltpu.async_copy(
                bufs[b], out_hbm.at[pl.ds(base + c * _CHUNK, _CHUNK)],
                wsems[b])

        def wait_writeback(b):
            pltpu.make_async_copy(
                bufs[b], out_hbm.at[pl.ds(base, _CHUNK)], wsems[b]).wait()

        for b in range(_NBUF - 1):
            gather(b, b)

        @pl.loop(0, n_chunks, step=_NBUF)
        def body(i):
            for b in range(_NBUF):
                c = i + b
                pb = (b + _NBUF - 1) % _NBUF
                pc = c + _NBUF - 1

                @pl.when(pc < n_chunks)
                def _():
                    @pl.when(pc >= _NBUF)
                    def _():
                        wait_writeback(pb)
                    gather(pc, pb)

                wait_gather(b)
                writeback(c, b)

        for b in range(_NBUF):
            wait_writeback(b)

    return k(table, ids_flat)


def kernel(input_ids, embed_weight):
    b, s = input_ids.shape
    vocab, hidden = embed_weight.shape
    ids_flat = input_ids.reshape(-1).astype(jnp.int32)
    out = _embed_lookup(ids_flat, embed_weight,
                        n_tokens=b * s, hidden=hidden)
    return out.reshape(b, s, hidden)
